# Initial kernel scaffold; baseline (speedup 1.0000x reference)
import jax
import jax.numpy as jnp
from jax.experimental import pallas as pl


def kernel(logits, labels, category_labels):
    raise NotImplementedError("write your pallas kernel here")


# speedup vs baseline: 18.1862x; 18.1862x over previous
import functools

import jax
import jax.numpy as jnp
from jax import lax
from jax.experimental import pallas as pl
from jax.experimental.pallas import tpu as pltpu
from jax.experimental.pallas import tpu_sc as plsc

_NUM_CATEGORIES = 8
_CMDS_PER_CAT = 16
_NUM_COMMANDS = _NUM_CATEGORIES * _CMDS_PER_CAT
_BATCH = 16384
_LANES = 16
_NUM_WORKERS = 32
_ROWS_PER_TILE = _BATCH // _NUM_WORKERS
_GROUPS_PER_TILE = _ROWS_PER_TILE // _LANES

_LN2 = 0.6931471805599453
_SQRT2 = 1.4142135623730951


def _ln(v):
    bits = plsc.bitcast(v, jnp.int32)
    e = lax.shift_right_arithmetic(bits, 23) - 127
    mant = plsc.bitcast(
        jnp.bitwise_or(jnp.bitwise_and(bits, 0x7FFFFF), 0x3F800000), jnp.float32
    )
    big = mant > jnp.float32(_SQRT2)
    mant = jnp.where(big, mant * jnp.float32(0.5), mant)
    e = e + jnp.where(big, 1, 0)
    z = (mant - 1.0) / (mant + 1.0)
    z2 = z * z
    p = jnp.full((_LANES,), 1.0 / 9.0, jnp.float32)
    for c in (1.0 / 7.0, 1.0 / 5.0, 1.0 / 3.0, 1.0):
        p = p * z2 + jnp.float32(c)
    return e.astype(jnp.float32) * jnp.float32(_LN2) + 2.0 * z * p


def _tile_body(logits_hbm, labels_hbm, cats_hbm, out_hbm,
               slab, lab_v, cat_v, catbuf, accbuf):
    nc = 2
    wid = lax.axis_index("s") * nc + lax.axis_index("c")
    base = wid * _ROWS_PER_TILE

    pltpu.sync_copy(
        logits_hbm.at[pl.ds(base * _NUM_COMMANDS, _ROWS_PER_TILE * _NUM_COMMANDS)],
        slab)
    pltpu.sync_copy(labels_hbm.at[pl.ds(base, _ROWS_PER_TILE)], lab_v)
    pltpu.sync_copy(cats_hbm.at[pl.ds(base, _ROWS_PER_TILE)], cat_v)

    iota = lax.iota(jnp.int32, _LANES)
    zeros = jnp.zeros((_LANES,), jnp.float32)
    neg_inf = jnp.full((_LANES,), -jnp.inf, jnp.float32)

    def group(g, carry):
        acc_cmd, acc_cat = carry
        r0 = g * _LANES
        rows = r0 + iota
        rows_off = rows * _NUM_COMMANDS

        m = [neg_inf] * 5
        for j in range(_NUM_COMMANDS):
            t = plsc.load_gather(slab, [rows_off + j])
            for si in range(5):
                hi = jnp.maximum(m[si], t)
                t = jnp.minimum(m[si], t)
                m[si] = hi
        m0 = m[0]
        t5 = m[4]

        s_parts = [zeros, zeros, zeros, zeros]
        cat_parts = [[zeros, zeros] for _ in range(_NUM_CATEGORIES)]
        for j in range(_NUM_COMMANDS):
            v = plsc.load_gather(slab, [rows_off + j])
            s_parts[j % 4] = s_parts[j % 4] + jnp.exp(v - m0)
            contrib = jnp.where(v >= t5, v, 0.0)
            c = j // _CMDS_PER_CAT
            cat_parts[c][j % 2] = cat_parts[c][j % 2] + contrib

        ssum = (s_parts[0] + s_parts[1]) + (s_parts[2] + s_parts[3])
        lse_cmd = m0 + _ln(ssum)
        labv = lab_v[pl.ds(r0, _LANES)]
        x_lab = plsc.load_gather(slab, [rows_off + labv])
        acc_cmd = acc_cmd + (lse_cmd - x_lab)

        cat = [cat_parts[c][0] + cat_parts[c][1] for c in range(_NUM_CATEGORIES)]
        cmax = jnp.maximum(jnp.maximum(jnp.maximum(cat[0], cat[1]),
                                       jnp.maximum(cat[2], cat[3])),
                           jnp.maximum(jnp.maximum(cat[4], cat[5]),
                                       jnp.maximum(cat[6], cat[7])))
        se = zeros
        for c in range(_NUM_CATEGORIES):
            se = se + jnp.exp(cat[c] - cmax)
            catbuf[pl.ds(c * _LANES, _LANES)] = cat[c]
        lse_cat = cmax + _ln(se)
        clabv = cat_v[pl.ds(r0, _LANES)]
        x_cat = plsc.load_gather(catbuf, [clabv * _LANES + iota])
        acc_cat = acc_cat + (lse_cat - x_cat)
        return acc_cmd, acc_cat

    acc_cmd, acc_cat = lax.fori_loop(0, _GROUPS_PER_TILE, group, (zeros, zeros))
    accbuf[...] = 0.6 * acc_cmd + 0.4 * acc_cat
    pltpu.sync_copy(accbuf, out_hbm.at[wid])


@jax.jit
def kernel(logits, labels, category_labels):
    mesh = plsc.VectorSubcoreMesh(core_axis_name="c", subcore_axis_name="s")
    parts = pl.kernel(
        _tile_body,
        out_type=jax.ShapeDtypeStruct((_NUM_WORKERS, _LANES), jnp.float32),
        mesh=mesh,
        compiler_params=pltpu.CompilerParams(needs_layout_passes=False),
        scratch_types=[
            pltpu.VMEM((_ROWS_PER_TILE * _NUM_COMMANDS,), jnp.float32),
            pltpu.VMEM((_ROWS_PER_TILE,), jnp.int32),
            pltpu.VMEM((_ROWS_PER_TILE,), jnp.int32),
            pltpu.VMEM((_NUM_CATEGORIES * _LANES,), jnp.float32),
            pltpu.VMEM((_LANES,), jnp.float32),
        ],
    )(logits.reshape(-1), labels, category_labels)
    return jnp.sum(parts) * jnp.float32(1.0 / _BATCH)
